# baseline (speedup 1.0000x reference)
import functools

import jax
import jax.numpy as jnp
from jax import lax
from jax.experimental import pallas as pl
from jax.experimental.pallas import tpu as pltpu

_LANE = 128


def _round_up(x, m):
    return (x + m - 1) // m * m


def _pick_row_tile(h, target=8):
    t = min(h, target)
    while h % t:
        t -= 1
    return t


def _conv_stats_kernel(xpad_ref, w_ref, ident_ref, convt_ref, stats_ref, *,
                       TH, Wo, Cinp, Coutp, KH, KW, dilation):
    h = pl.program_id(1)
    row0 = pl.multiple_of(h * TH, TH)

    patches = []
    for kh in range(KH):
        r0 = row0 + kh * dilation
        for kw in range(KW):
            c0 = kw * dilation
            p = xpad_ref[pl.ds(r0, TH), pl.ds(c0, Wo), :]
            patches.append(p.reshape(TH * Wo, Cinp))
    lhs = jnp.concatenate(patches, axis=-1)

    acc = jnp.dot(lhs, w_ref[...], preferred_element_type=jnp.float32)

    s = jnp.sum(acc, axis=0, keepdims=True)
    ss = jnp.sum(acc * acc, axis=0, keepdims=True)
    stats_ref[...] = jnp.concatenate([s, ss], axis=0)

    accb = acc.astype(jnp.bfloat16)
    acct = lax.dot_general(accb, ident_ref[...],
                           (((0,), (0,)), ((), ())),
                           preferred_element_type=jnp.float32)
    convt_ref[...] = acct.astype(jnp.bfloat16)


def _bn_relu_kernel(stats_ref, gamma_ref, beta_ref, convt_ref, out_ref, *,
                    cnt, eps):
    tot = jnp.sum(stats_ref[...], axis=0)
    mean = tot[0:1, :] / cnt
    var = jnp.maximum(tot[1:2, :] / cnt - mean * mean, 0.0)
    inv = lax.rsqrt(var + eps)
    scale_r = gamma_ref[...] * inv
    shift_r = beta_ref[...] - mean * scale_r
    scale = jnp.transpose(scale_r)
    shift = jnp.transpose(shift_r)

    y = convt_ref[...].astype(jnp.float32) * scale + shift
    out_ref[...] = jnp.maximum(y, 0.0).astype(out_ref.dtype)


def kernel(x_nchw, weight_oihw, gamma, beta):
    padding, dilation, eps = 2, 2, 1e-5
    N, Cin, H, W = x_nchw.shape
    Cout, _, KH, KW = weight_oihw.shape

    Ho = H + 2 * padding - dilation * (KH - 1)
    Wo = W + 2 * padding - dilation * (KW - 1)
    Hp, Wp = H + 2 * padding, W + 2 * padding

    Cinp = _round_up(Cin, _LANE)
    Coutp = _round_up(Cout, _LANE)

    x_nhwc = jnp.transpose(x_nchw, (0, 2, 3, 1))
    xpad = jnp.pad(x_nhwc, ((0, 0), (padding, padding), (padding, padding),
                            (0, Cinp - Cin))).astype(jnp.bfloat16)

    w = jnp.transpose(weight_oihw, (2, 3, 1, 0))
    w = jnp.pad(w, ((0, 0), (0, 0), (0, Cinp - Cin), (0, Coutp - Cout)))
    w2d = w.reshape(KH * KW * Cinp, Coutp).astype(jnp.bfloat16)

    TH = _pick_row_tile(Ho, target=8)
    Hg = Ho // TH
    P = TH * Wo

    ident = jnp.eye(P, dtype=jnp.bfloat16)

    conv_kernel = functools.partial(
        _conv_stats_kernel, TH=TH, Wo=Wo, Cinp=Cinp, Coutp=Coutp,
        KH=KH, KW=KW, dilation=dilation)

    convt, stats = pl.pallas_call(
        conv_kernel,
        out_shape=(jax.ShapeDtypeStruct((N, Coutp, Ho * Wo), jnp.bfloat16),
                   jax.ShapeDtypeStruct((N, Hg, 2, Coutp), jnp.float32)),
        grid=(N, Hg),
        in_specs=[
            pl.BlockSpec((None, Hp, Wp, Cinp), lambda n, h: (n, 0, 0, 0)),
            pl.BlockSpec((KH * KW * Cinp, Coutp), lambda n, h: (0, 0)),
            pl.BlockSpec((P, P), lambda n, h: (0, 0)),
        ],
        out_specs=(
            pl.BlockSpec((None, Coutp, P), lambda n, h: (n, 0, h)),
            pl.BlockSpec((None, None, 2, Coutp), lambda n, h: (n, h, 0, 0)),
        ),
        compiler_params=pltpu.CompilerParams(
            dimension_semantics=("parallel", "parallel"),
            vmem_limit_bytes=32 * 1024 * 1024),
    )(xpad, w2d, ident)

    gamma_r = jnp.pad(gamma.astype(jnp.float32), (0, Coutp - Cout))
    beta_r = jnp.pad(beta.astype(jnp.float32), (0, Coutp - Cout))
    gamma_r = gamma_r.reshape(1, Coutp)
    beta_r = beta_r.reshape(1, Coutp)
    stats2 = stats.reshape(N * Hg, 2, Coutp)

    JB = 4
    PJ = Ho * Wo // JB

    bn_kernel = functools.partial(_bn_relu_kernel,
                                  cnt=float(N * Ho * Wo), eps=eps)

    out = pl.pallas_call(
        bn_kernel,
        out_shape=jax.ShapeDtypeStruct((N, Coutp, Ho * Wo), jnp.float32),
        grid=(N, JB),
        in_specs=[
            pl.BlockSpec((N * Hg, 2, Coutp), lambda n, j: (0, 0, 0)),
            pl.BlockSpec((1, Coutp), lambda n, j: (0, 0)),
            pl.BlockSpec((1, Coutp), lambda n, j: (0, 0)),
            pl.BlockSpec((None, Coutp, PJ), lambda n, j: (n, 0, j)),
        ],
        out_specs=pl.BlockSpec((None, Coutp, PJ), lambda n, j: (n, 0, j)),
        compiler_params=pltpu.CompilerParams(
            dimension_semantics=("parallel", "parallel")),
    )(stats2, gamma_r, beta_r, convt)

    return out[:, :Cout, :].reshape(N, Cout, Ho, Wo)


# speedup vs baseline: 1.0276x; 1.0276x over previous
import functools

import jax
import jax.numpy as jnp
from jax import lax
from jax.experimental import pallas as pl
from jax.experimental.pallas import tpu as pltpu

_LANE = 128


def _round_up(x, m):
    return (x + m - 1) // m * m


def _conv_stats_kernel(xpad_ref, w_ref, ident_ref, convt_ref, stats_ref,
                       shifted_ref, *, TH, Hg, Wo, Cinp, Coutp, KH, KW,
                       dilation):
    for kw in range(KW):
        shifted_ref[:, :, kw * Cinp:(kw + 1) * Cinp] = (
            xpad_ref[:, pl.ds(kw * dilation, Wo), :])

    P = TH * Wo
    s_acc = jnp.zeros((1, Coutp), jnp.float32)
    ss_acc = jnp.zeros((1, Coutp), jnp.float32)
    for sidx in range(Hg):
        row0 = sidx * TH
        acc = jnp.zeros((P, Coutp), jnp.float32)
        for kh in range(KH):
            lhs = shifted_ref[pl.ds(row0 + kh * dilation, TH), :, :]
            lhs = lhs.reshape(P, KW * Cinp)
            w_kh = w_ref[pl.ds(kh * KW * Cinp, KW * Cinp), :]
            acc = acc + jnp.dot(lhs, w_kh,
                                preferred_element_type=jnp.float32)

        s_acc = s_acc + jnp.sum(acc, axis=0, keepdims=True)
        ss_acc = ss_acc + jnp.sum(acc * acc, axis=0, keepdims=True)

        accb = acc.astype(jnp.bfloat16)
        acct = lax.dot_general(accb, ident_ref[...],
                               (((0,), (0,)), ((), ())),
                               preferred_element_type=jnp.float32)
        convt_ref[:, pl.ds(sidx * P, P)] = acct.astype(jnp.bfloat16)

    stats_ref[...] = jnp.concatenate([s_acc, ss_acc], axis=0)


def _bn_relu_kernel(stats_ref, gamma_ref, beta_ref, convt_ref, out_ref, *,
                    cnt, eps):
    tot = jnp.sum(stats_ref[...], axis=0)
    mean = tot[0:1, :] / cnt
    var = jnp.maximum(tot[1:2, :] / cnt - mean * mean, 0.0)
    inv = lax.rsqrt(var + eps)
    scale_r = gamma_ref[...] * inv
    shift_r = beta_ref[...] - mean * scale_r
    scale = jnp.transpose(scale_r)
    shift = jnp.transpose(shift_r)

    y = convt_ref[...].astype(jnp.float32) * scale + shift
    out_ref[...] = jnp.maximum(y, 0.0).astype(out_ref.dtype)


def kernel(x_nchw, weight_oihw, gamma, beta):
    padding, dilation, eps = 2, 2, 1e-5
    N, Cin, H, W = x_nchw.shape
    Cout, _, KH, KW = weight_oihw.shape

    Ho = H + 2 * padding - dilation * (KH - 1)
    Wo = W + 2 * padding - dilation * (KW - 1)
    Hp, Wp = H + 2 * padding, W + 2 * padding

    Cinp = _round_up(Cin, _LANE)
    Coutp = _round_up(Cout, _LANE)

    x_nhwc = jnp.transpose(x_nchw, (0, 2, 3, 1))
    xpad = jnp.pad(x_nhwc, ((0, 0), (padding, padding), (padding, padding),
                            (0, Cinp - Cin))).astype(jnp.bfloat16)

    w = jnp.transpose(weight_oihw, (2, 3, 1, 0))
    w = jnp.pad(w, ((0, 0), (0, 0), (0, Cinp - Cin), (0, Coutp - Cout)))
    w2d = w.reshape(KH * KW * Cinp, Coutp).astype(jnp.bfloat16)

    TH = 8
    Hg = Ho // TH
    P = TH * Wo

    ident = jnp.eye(P, dtype=jnp.bfloat16)

    conv_kernel = functools.partial(
        _conv_stats_kernel, TH=TH, Hg=Hg, Wo=Wo, Cinp=Cinp, Coutp=Coutp,
        KH=KH, KW=KW, dilation=dilation)

    convt, stats = pl.pallas_call(
        conv_kernel,
        out_shape=(jax.ShapeDtypeStruct((N, Coutp, Ho * Wo), jnp.bfloat16),
                   jax.ShapeDtypeStruct((N, 2, Coutp), jnp.float32)),
        grid=(N,),
        in_specs=[
            pl.BlockSpec((None, Hp, Wp, Cinp), lambda n: (n, 0, 0, 0)),
            pl.BlockSpec((KH * KW * Cinp, Coutp), lambda n: (0, 0)),
            pl.BlockSpec((P, P), lambda n: (0, 0)),
        ],
        out_specs=(
            pl.BlockSpec((None, Coutp, Ho * Wo), lambda n: (n, 0, 0)),
            pl.BlockSpec((None, 2, Coutp), lambda n: (n, 0, 0)),
        ),
        scratch_shapes=[pltpu.VMEM((Hp, Wo, KW * Cinp), jnp.bfloat16)],
        compiler_params=pltpu.CompilerParams(
            dimension_semantics=("parallel",),
            vmem_limit_bytes=32 * 1024 * 1024),
    )(xpad, w2d, ident)

    gamma_r = jnp.pad(gamma.astype(jnp.float32), (0, Coutp - Cout))
    beta_r = jnp.pad(beta.astype(jnp.float32), (0, Coutp - Cout))
    gamma_r = gamma_r.reshape(1, Coutp)
    beta_r = beta_r.reshape(1, Coutp)

    bn_kernel = functools.partial(_bn_relu_kernel,
                                  cnt=float(N * Ho * Wo), eps=eps)

    out = pl.pallas_call(
        bn_kernel,
        out_shape=jax.ShapeDtypeStruct((N, Coutp, Ho * Wo), jnp.float32),
        grid=(N,),
        in_specs=[
            pl.BlockSpec((N, 2, Coutp), lambda n: (0, 0, 0)),
            pl.BlockSpec((1, Coutp), lambda n: (0, 0)),
            pl.BlockSpec((1, Coutp), lambda n: (0, 0)),
            pl.BlockSpec((None, Coutp, Ho * Wo), lambda n: (n, 0, 0)),
        ],
        out_specs=pl.BlockSpec((None, Coutp, Ho * Wo), lambda n: (n, 0, 0)),
        compiler_params=pltpu.CompilerParams(
            dimension_semantics=("parallel",)),
    )(stats, gamma_r, beta_r, convt)

    return out[:, :Cout, :].reshape(N, Cout, Ho, Wo)


# speedup vs baseline: 1.5225x; 1.4816x over previous
import functools

import jax
import jax.numpy as jnp
from jax import lax
from jax.experimental import pallas as pl
from jax.experimental.pallas import tpu as pltpu

_LANE = 128


def _round_up(x, m):
    return (x + m - 1) // m * m


def _conv_stats_kernel(xpad_ref, w_ref, convt_ref, stats_ref,
                       shifted_ref, *, TH, Hg, Wo, Cinp, Coutp, KH, KW,
                       dilation):
    for kw in range(KW):
        shifted_ref[:, :, kw * Cinp:(kw + 1) * Cinp] = (
            xpad_ref[:, pl.ds(kw * dilation, Wo), :])

    P = TH * Wo
    halo = dilation * (KH - 1)
    s_acc = jnp.zeros((1, Coutp), jnp.float32)
    ss_acc = jnp.zeros((1, Coutp), jnp.float32)
    for sidx in range(Hg):
        row0 = sidx * TH
        region = shifted_ref[pl.ds(row0, TH + halo), :, :]
        acc = jnp.zeros((P, Coutp), jnp.float32)
        for kh in range(KH):
            lhs = region[kh * dilation:kh * dilation + TH]
            lhs = lhs.reshape(P, KW * Cinp)
            w_kh = w_ref[pl.ds(kh * KW * Cinp, KW * Cinp), :]
            acc = acc + jnp.dot(lhs, w_kh,
                                preferred_element_type=jnp.float32)

        s_acc = s_acc + jnp.sum(acc, axis=0, keepdims=True)
        ss_acc = ss_acc + jnp.sum(acc * acc, axis=0, keepdims=True)

        accb = acc.astype(jnp.bfloat16)
        acct = jnp.transpose(accb)
        convt_ref[:, pl.ds(sidx * P, P)] = acct

    stats_ref[...] = jnp.concatenate([s_acc, ss_acc], axis=0)


def _bn_relu_kernel(stats_ref, gamma_ref, beta_ref, convt_ref, out_ref, *,
                    cnt, eps):
    tot = jnp.sum(stats_ref[...], axis=0)
    mean = tot[0:1, :] / cnt
    var = jnp.maximum(tot[1:2, :] / cnt - mean * mean, 0.0)
    inv = lax.rsqrt(var + eps)
    scale_r = gamma_ref[...] * inv
    shift_r = beta_ref[...] - mean * scale_r
    scale = jnp.transpose(scale_r)
    shift = jnp.transpose(shift_r)

    y = convt_ref[...].astype(jnp.float32) * scale + shift
    out_ref[...] = jnp.maximum(y, 0.0).astype(out_ref.dtype)


def kernel(x_nchw, weight_oihw, gamma, beta):
    padding, dilation, eps = 2, 2, 1e-5
    N, Cin, H, W = x_nchw.shape
    Cout, _, KH, KW = weight_oihw.shape

    Ho = H + 2 * padding - dilation * (KH - 1)
    Wo = W + 2 * padding - dilation * (KW - 1)
    Hp, Wp = H + 2 * padding, W + 2 * padding

    Cinp = _round_up(Cin, _LANE)
    Coutp = _round_up(Cout, _LANE)

    x_nhwc = jnp.transpose(x_nchw, (0, 2, 3, 1))
    xpad = jnp.pad(x_nhwc, ((0, 0), (padding, padding), (padding, padding),
                            (0, Cinp - Cin))).astype(jnp.bfloat16)

    w = jnp.transpose(weight_oihw, (2, 3, 1, 0))
    w = jnp.pad(w, ((0, 0), (0, 0), (0, Cinp - Cin), (0, Coutp - Cout)))
    w2d = w.reshape(KH * KW * Cinp, Coutp).astype(jnp.bfloat16)

    TH = 8
    Hg = Ho // TH
    P = TH * Wo

    conv_kernel = functools.partial(
        _conv_stats_kernel, TH=TH, Hg=Hg, Wo=Wo, Cinp=Cinp, Coutp=Coutp,
        KH=KH, KW=KW, dilation=dilation)

    convt, stats = pl.pallas_call(
        conv_kernel,
        out_shape=(jax.ShapeDtypeStruct((N, Coutp, Ho * Wo), jnp.bfloat16),
                   jax.ShapeDtypeStruct((N, 2, Coutp), jnp.float32)),
        grid=(N,),
        in_specs=[
            pl.BlockSpec((None, Hp, Wp, Cinp), lambda n: (n, 0, 0, 0)),
            pl.BlockSpec((KH * KW * Cinp, Coutp), lambda n: (0, 0)),
        ],
        out_specs=(
            pl.BlockSpec((None, Coutp, Ho * Wo), lambda n: (n, 0, 0)),
            pl.BlockSpec((None, 2, Coutp), lambda n: (n, 0, 0)),
        ),
        scratch_shapes=[pltpu.VMEM((Hp, Wo, KW * Cinp), jnp.bfloat16)],
        compiler_params=pltpu.CompilerParams(
            dimension_semantics=("parallel",),
            vmem_limit_bytes=32 * 1024 * 1024),
    )(xpad, w2d)

    gamma_r = jnp.pad(gamma.astype(jnp.float32), (0, Coutp - Cout))
    beta_r = jnp.pad(beta.astype(jnp.float32), (0, Coutp - Cout))
    gamma_r = gamma_r.reshape(1, Coutp)
    beta_r = beta_r.reshape(1, Coutp)

    bn_kernel = functools.partial(_bn_relu_kernel,
                                  cnt=float(N * Ho * Wo), eps=eps)

    out = pl.pallas_call(
        bn_kernel,
        out_shape=jax.ShapeDtypeStruct((N, Coutp, Ho * Wo), jnp.float32),
        grid=(N,),
        in_specs=[
            pl.BlockSpec((N, 2, Coutp), lambda n: (0, 0, 0)),
            pl.BlockSpec((1, Coutp), lambda n: (0, 0)),
            pl.BlockSpec((1, Coutp), lambda n: (0, 0)),
            pl.BlockSpec((None, Coutp, Ho * Wo), lambda n: (n, 0, 0)),
        ],
        out_specs=pl.BlockSpec((None, Coutp, Ho * Wo), lambda n: (n, 0, 0)),
        compiler_params=pltpu.CompilerParams(
            dimension_semantics=("parallel",)),
    )(stats, gamma_r, beta_r, convt)

    return out[:, :Cout, :].reshape(N, Cout, Ho, Wo)
